# baseline (speedup 1.0000x reference)
import functools

import jax
import jax.numpy as jnp
from jax import lax
from jax.experimental import pallas as pl
from jax.experimental.pallas import tpu as pltpu
from jax.experimental.pallas import tpu_sc as plsc

N = 8192
D_CHL = 16
D_MODEL = 64
D_PAD = 128
N_HEADS = 4
HEAD_DIM = D_MODEL // N_HEADS
D_FFN = 128
M = 16
BIG = 1 << 29

ROWS_B = 512
ROWS_D = 512


def _src_body(feat_ref, crd_ref, wp_ref, bp_ref, w1_ref, b1_ref, w2_ref,
              b2_ref, out_ref):
    nc = crd_ref[...] * (1.0 / 399.0)
    pe = jnp.maximum(
        jnp.dot(nc, w1_ref[...], preferred_element_type=jnp.float32)
        + b1_ref[...], 0.0)
    pe = jnp.dot(pe, w2_ref[...], preferred_element_type=jnp.float32) + b2_ref[...]
    proj = jnp.dot(feat_ref[...], wp_ref[...],
                   preferred_element_type=jnp.float32) + bp_ref[...]
    out_ref[...] = proj + pe


def _compute_src(feat, crd_f32, W_proj, b_proj, W_pe1, b_pe1, W_pe2, b_pe2):
    W_proj_p = jnp.pad(W_proj, ((0, 0), (0, D_PAD - D_MODEL)))
    b_proj_p = jnp.pad(b_proj, (0, D_PAD - D_MODEL))
    W_pe2_p = jnp.pad(W_pe2, ((0, 0), (0, D_PAD - D_MODEL)))
    b_pe2_p = jnp.pad(b_pe2, (0, D_PAD - D_MODEL))
    full = lambda s: pl.BlockSpec(s, lambda: (0,) * len(s))
    return pl.pallas_call(
        _src_body,
        out_shape=jax.ShapeDtypeStruct((N, D_PAD), jnp.float32),
        in_specs=[full((N, D_CHL)), full((N, 3)), full((D_CHL, D_PAD)),
                  full((1, D_PAD)), full((3, D_MODEL // 2)),
                  full((1, D_MODEL // 2)), full((D_MODEL // 2, D_PAD)),
                  full((1, D_PAD))],
        out_specs=full((N, D_PAD)),
    )(feat, crd_f32, W_proj_p, b_proj_p.reshape(1, -1), W_pe1,
      b_pe1.reshape(1, -1), W_pe2_p, b_pe2_p.reshape(1, -1))


def _topk_body(xr_ref, yr_ref, zr_ref, xc_ref, yc_ref, zc_ref,
               idx_ref, msk_ref, mprev_ref):
    dx = jnp.abs(xr_ref[...] - xc_ref[...])
    dy = jnp.abs(yr_ref[...] - yc_ref[...])
    dz = jnp.abs(zr_ref[...] - zc_ref[...])
    man = dx + dy + dz
    col = lax.broadcasted_iota(jnp.int32, (ROWS_B, N), 1)
    valid_e = man <= 2
    key = jnp.where(valid_e, man * N + col, BIG)
    rowid = (pl.program_id(0) * ROWS_B
             + lax.broadcasted_iota(jnp.int32, (ROWS_B, 1), 0))
    cntmax = jnp.max(jnp.sum(valid_e.astype(jnp.int32), axis=1))
    idx_ref[...] = jnp.broadcast_to(rowid, (ROWS_B, M))
    msk_ref[...] = jnp.zeros((ROWS_B, M), jnp.float32)
    mprev_ref[...] = jnp.full((ROWS_B, 1), -1, jnp.int32)
    for t in range(M):
        @pl.when(t < cntmax)
        def _():
            cand = jnp.where(key > mprev_ref[...], key, BIG)
            m = jnp.min(cand, axis=1, keepdims=True)
            valid = m < 3 * N
            idx_ref[:, t:t + 1] = jnp.where(
                valid, lax.bitwise_and(m, N - 1), rowid)
            msk_ref[:, t:t + 1] = valid.astype(jnp.float32)
            mprev_ref[...] = m


def _topk(crd_i32):
    crd = crd_i32
    xr = crd[:, 0:1]
    yr = crd[:, 1:2]
    zr = crd[:, 2:3]
    xc = crd[:, 0].reshape(1, N)
    yc = crd[:, 1].reshape(1, N)
    zc = crd[:, 2].reshape(1, N)
    rspec = pl.BlockSpec((ROWS_B, 1), lambda i: (i, 0))
    cspec = pl.BlockSpec((1, N), lambda i: (0, 0))
    ospec = pl.BlockSpec((ROWS_B, M), lambda i: (i, 0))
    return pl.pallas_call(
        _topk_body,
        grid=(N // ROWS_B,),
        out_shape=[jax.ShapeDtypeStruct((N, M), jnp.int32),
                   jax.ShapeDtypeStruct((N, M), jnp.float32)],
        in_specs=[rspec, rspec, rspec, cspec, cspec, cspec],
        out_specs=[ospec, ospec],
        scratch_shapes=[pltpu.VMEM((ROWS_B, 1), jnp.int32)],
    )(xr, yr, zr, xc, yc, zc)


B_TOT = N * M + N
_NW = 32
_PER_W = B_TOT // _NW
_CH = 544


def _sc_gather(table, idx_all):
    info = plsc.get_sparse_core_info()
    nc = info.num_cores
    mesh = plsc.VectorSubcoreMesh(core_axis_name="c", subcore_axis_name="s")

    @functools.partial(
        pl.kernel, mesh=mesh,
        out_type=jax.ShapeDtypeStruct((B_TOT, D_PAD), jnp.float32),
        scratch_types=[
            pltpu.VMEM((_CH,), jnp.int32),
            pltpu.VMEM((_CH, D_PAD), jnp.float32),
            pltpu.SemaphoreType.DMA,
        ],
    )
    def gk(idx_hbm, tab_hbm, out_hbm, idx_v, rows_v, sem):
        wid = lax.axis_index("s") * nc + lax.axis_index("c")
        for c in range(_PER_W // _CH):
            off = wid * _PER_W + c * _CH
            pltpu.sync_copy(idx_hbm.at[pl.ds(off, _CH)], idx_v)
            pltpu.async_copy(tab_hbm.at[idx_v], rows_v, sem).wait()
            pltpu.sync_copy(rows_v, out_hbm.at[pl.ds(off, _CH)])

    return gk(idx_all, table)


def _attn_body(q_ref, mq_ref, kv_ref, mkv_ref, feat_ref,
               wq_ref, bq_ref, wk_ref, bk_ref, wv_ref, bv_ref,
               wo_ref, bo_ref, wf1_ref, bf1_ref, wf2_ref, bf2_ref,
               lng_ref, lnb_ref, wfu1_ref, bfu1_ref,
               wfu2a_ref, wfu2b_ref, bfu2_ref, out_ref):
    qp = (mq_ref[...] * jnp.dot(q_ref[...], wq_ref[...],
                                preferred_element_type=jnp.float32)
          + bq_ref[...])
    kvf = kv_ref[...].reshape(M * ROWS_D, D_PAD)
    mkv = mkv_ref[...][:, :, None]
    b3 = lambda r: r[...].reshape(1, 1, D_MODEL)
    kp = mkv * jnp.dot(kvf, wk_ref[...],
                       preferred_element_type=jnp.float32).reshape(
                           M, ROWS_D, D_MODEL) + b3(bk_ref)
    vp = mkv * jnp.dot(kvf, wv_ref[...],
                       preferred_element_type=jnp.float32).reshape(
                           M, ROWS_D, D_MODEL) + b3(bv_ref)
    prod = kp * qp[None, :, :]
    lane = lax.broadcasted_iota(jnp.int32, (1, 1, D_MODEL), 2)
    wfull = jnp.zeros((M, ROWS_D, D_MODEL), jnp.float32)
    for h in range(N_HEADS):
        mh = ((lane // HEAD_DIM) == h).astype(jnp.float32)
        s = jnp.sum(prod * mh, axis=-1) * (1.0 / 4.0)
        s = s - jnp.max(s, axis=0, keepdims=True)
        e = jnp.exp(s)
        w = e / jnp.sum(e, axis=0, keepdims=True)
        wfull = wfull + w[:, :, None] * mh
    attn = jnp.sum(wfull * vp, axis=0)
    out = jnp.dot(attn, wo_ref[...], preferred_element_type=jnp.float32) + bo_ref[...]
    ff = jnp.maximum(
        jnp.dot(out, wf1_ref[...], preferred_element_type=jnp.float32)
        + bf1_ref[...], 0.0)
    ff = jnp.dot(ff, wf2_ref[...], preferred_element_type=jnp.float32) + bf2_ref[...]
    t = out + ff
    mu = jnp.mean(t, axis=-1, keepdims=True)
    var = jnp.mean((t - mu) * (t - mu), axis=-1, keepdims=True)
    tgt = (t - mu) * lax.rsqrt(var + 1e-5) * lng_ref[...] + lnb_ref[...]
    tf = jnp.dot(tgt, wfu1_ref[...], preferred_element_type=jnp.float32) + bfu1_ref[...]
    fused = (jnp.dot(feat_ref[...], wfu2a_ref[...],
                     preferred_element_type=jnp.float32)
             + jnp.dot(tf, wfu2b_ref[...], preferred_element_type=jnp.float32)
             + bfu2_ref[...])
    out_ref[...] = fused


def _attention(Q, maskQ, KV3, maskKV, feat, Wq, bq, Wk, bk, Wv, bv, Wo, bo,
               W_f1, b_f1, W_f2, b_f2, ln_g, ln_b, W_fu1, b_fu1,
               W_fu2, b_fu2):
    Wq_p = jnp.pad(Wq, ((0, D_PAD - D_MODEL), (0, 0)))
    Wk_p = jnp.pad(Wk, ((0, D_PAD - D_MODEL), (0, 0)))
    Wv_p = jnp.pad(Wv, ((0, D_PAD - D_MODEL), (0, 0)))
    row2 = lambda w: pl.BlockSpec((ROWS_D, w), lambda i: (i, 0))
    fullw = lambda s: pl.BlockSpec(s, lambda i: (0,) * len(s))
    kvspec = pl.BlockSpec((M, ROWS_D, D_PAD), lambda i: (0, i, 0))
    mkvspec = pl.BlockSpec((M, ROWS_D), lambda i: (0, i))
    return pl.pallas_call(
        _attn_body,
        grid=(N // ROWS_D,),
        out_shape=jax.ShapeDtypeStruct((N, D_CHL), jnp.float32),
        in_specs=[row2(D_PAD), row2(1), kvspec, mkvspec, row2(D_CHL),
                  fullw((D_PAD, D_MODEL)), fullw((1, D_MODEL)),
                  fullw((D_PAD, D_MODEL)), fullw((1, D_MODEL)),
                  fullw((D_PAD, D_MODEL)), fullw((1, D_MODEL)),
                  fullw((D_MODEL, D_MODEL)), fullw((1, D_MODEL)),
                  fullw((D_MODEL, D_FFN)), fullw((1, D_FFN)),
                  fullw((D_FFN, D_MODEL)), fullw((1, D_MODEL)),
                  fullw((1, D_MODEL)), fullw((1, D_MODEL)),
                  fullw((D_MODEL, D_CHL)), fullw((1, D_CHL)),
                  fullw((D_CHL, D_CHL)), fullw((D_CHL, D_CHL)),
                  fullw((1, D_CHL))],
        out_specs=row2(D_CHL),
    )(Q, maskQ, KV3, maskKV, feat,
      Wq_p, bq.reshape(1, -1), Wk_p, bk.reshape(1, -1), Wv_p, bv.reshape(1, -1),
      Wo, bo.reshape(1, -1), W_f1, b_f1.reshape(1, -1), W_f2,
      b_f2.reshape(1, -1), ln_g.reshape(1, -1), ln_b.reshape(1, -1),
      W_fu1, b_fu1.reshape(1, -1), W_fu2[:D_CHL], W_fu2[D_CHL:],
      b_fu2.reshape(1, -1))


def _bn_body(x_ref, g_ref, b_ref, out_ref):
    x = x_ref[...]
    mu = jnp.mean(x, axis=0, keepdims=True)
    var = jnp.mean((x - mu) * (x - mu), axis=0, keepdims=True)
    y = (x - mu) * lax.rsqrt(var + 1e-5) * g_ref[...] + b_ref[...]
    out_ref[...] = jnp.maximum(y, 0.0)


def _batchnorm(x, g, b):
    full = lambda s: pl.BlockSpec(s, lambda: (0,) * len(s))
    return pl.pallas_call(
        _bn_body,
        out_shape=jax.ShapeDtypeStruct((N, D_CHL), jnp.float32),
        in_specs=[full((N, D_CHL)), full((1, D_CHL)), full((1, D_CHL))],
        out_specs=full((N, D_CHL)),
    )(x, g.reshape(1, -1), b.reshape(1, -1))


def kernel(voxel_coords, voxel_features, W_proj, b_proj, W_pe1, b_pe1,
           W_pe2, b_pe2, Wq, bq, Wk, bk, Wv, bv, Wo, bo, W_f1, b_f1,
           W_f2, b_f2, ln_g, ln_b, W_fu1, b_fu1, W_fu2, b_fu2, bn_g, bn_b):
    crd_i32 = voxel_coords.astype(jnp.int32)
    crd_f32 = voxel_coords.astype(jnp.float32)

    src = _compute_src(voxel_features, crd_f32, W_proj, b_proj,
                       W_pe1, b_pe1, W_pe2, b_pe2)
    idx, msk = _topk(crd_i32)

    kv_idx = idx.reshape(M, N)
    kv_msk = msk.reshape(M, N)
    q_msk = msk[:, 0:1]

    idx_all = jnp.concatenate([kv_idx.reshape(-1), idx[:, 0]], axis=0)
    rows = _sc_gather(src, idx_all)
    KV3 = rows[:N * M].reshape(M, N, D_PAD)
    Q = rows[N * M:]

    fused = _attention(Q, q_msk, KV3, kv_msk, voxel_features,
                       Wq, bq, Wk, bk, Wv, bv, Wo, bo, W_f1, b_f1,
                       W_f2, b_f2, ln_g, ln_b, W_fu1, b_fu1, W_fu2, b_fu2)
    return _batchnorm(fused, bn_g, bn_b)


# speedup vs baseline: 1.6316x; 1.6316x over previous
import functools

import jax
import jax.numpy as jnp
from jax import lax
from jax.experimental import pallas as pl
from jax.experimental.pallas import tpu as pltpu
from jax.experimental.pallas import tpu_sc as plsc

N = 8192
D_CHL = 16
D_MODEL = 64
D_PAD = 128
N_HEADS = 4
HEAD_DIM = D_MODEL // N_HEADS
D_FFN = 128
M = 16
BIG = 1 << 29

ROWS_B = 128
ROWS_D = 512


def _src_body(feat_ref, crd_ref, wp_ref, bp_ref, w1_ref, b1_ref, w2_ref,
              b2_ref, out_ref):
    nc = crd_ref[...] * (1.0 / 399.0)
    pe = jnp.maximum(
        jnp.dot(nc, w1_ref[...], preferred_element_type=jnp.float32)
        + b1_ref[...], 0.0)
    pe = jnp.dot(pe, w2_ref[...], preferred_element_type=jnp.float32) + b2_ref[...]
    proj = jnp.dot(feat_ref[...], wp_ref[...],
                   preferred_element_type=jnp.float32) + bp_ref[...]
    out_ref[...] = proj + pe


def _compute_src(feat, crd_f32, W_proj, b_proj, W_pe1, b_pe1, W_pe2, b_pe2):
    W_proj_p = jnp.pad(W_proj, ((0, 0), (0, D_PAD - D_MODEL)))
    b_proj_p = jnp.pad(b_proj, (0, D_PAD - D_MODEL))
    W_pe2_p = jnp.pad(W_pe2, ((0, 0), (0, D_PAD - D_MODEL)))
    b_pe2_p = jnp.pad(b_pe2, (0, D_PAD - D_MODEL))
    full = lambda s: pl.BlockSpec(s, lambda: (0,) * len(s))
    return pl.pallas_call(
        _src_body,
        out_shape=jax.ShapeDtypeStruct((N, D_PAD), jnp.float32),
        in_specs=[full((N, D_CHL)), full((N, 3)), full((D_CHL, D_PAD)),
                  full((1, D_PAD)), full((3, D_MODEL // 2)),
                  full((1, D_MODEL // 2)), full((D_MODEL // 2, D_PAD)),
                  full((1, D_PAD))],
        out_specs=full((N, D_PAD)),
    )(feat, crd_f32, W_proj_p, b_proj_p.reshape(1, -1), W_pe1,
      b_pe1.reshape(1, -1), W_pe2_p, b_pe2_p.reshape(1, -1))


def _topk_body(xr_ref, yr_ref, zr_ref, xc_ref, yc_ref, zc_ref,
               idx_ref, msk_ref, mprev_ref):
    dx = jnp.abs(xr_ref[...] - xc_ref[...])
    dy = jnp.abs(yr_ref[...] - yc_ref[...])
    dz = jnp.abs(zr_ref[...] - zc_ref[...])
    man = dx + dy + dz
    col = lax.broadcasted_iota(jnp.int32, (ROWS_B, N), 1)
    valid_e = man <= 2
    key = jnp.where(valid_e, man * N + col, BIG)
    rowid = (pl.program_id(0) * ROWS_B
             + lax.broadcasted_iota(jnp.int32, (ROWS_B, 1), 0))
    cntmax = jnp.max(jnp.sum(valid_e.astype(jnp.int32), axis=1))
    idx_ref[...] = jnp.broadcast_to(rowid, (ROWS_B, M))
    msk_ref[...] = jnp.zeros((ROWS_B, M), jnp.float32)
    mprev_ref[...] = jnp.full((ROWS_B, 1), -1, jnp.int32)
    for t in range(M):
        @pl.when(t < cntmax)
        def _():
            cand = jnp.where(key > mprev_ref[...], key, BIG)
            m = jnp.min(cand, axis=1, keepdims=True)
            valid = m < 3 * N
            idx_ref[:, t:t + 1] = jnp.where(
                valid, lax.bitwise_and(m, N - 1), rowid)
            msk_ref[:, t:t + 1] = valid.astype(jnp.float32)
            mprev_ref[...] = m


def _topk(crd_i32):
    crd = crd_i32
    xr = crd[:, 0:1]
    yr = crd[:, 1:2]
    zr = crd[:, 2:3]
    xc = crd[:, 0].reshape(1, N)
    yc = crd[:, 1].reshape(1, N)
    zc = crd[:, 2].reshape(1, N)
    rspec = pl.BlockSpec((ROWS_B, 1), lambda i: (i, 0))
    cspec = pl.BlockSpec((1, N), lambda i: (0, 0))
    ospec = pl.BlockSpec((ROWS_B, M), lambda i: (i, 0))
    return pl.pallas_call(
        _topk_body,
        grid=(N // ROWS_B,),
        out_shape=[jax.ShapeDtypeStruct((N, M), jnp.int32),
                   jax.ShapeDtypeStruct((N, M), jnp.float32)],
        in_specs=[rspec, rspec, rspec, cspec, cspec, cspec],
        out_specs=[ospec, ospec],
        scratch_shapes=[pltpu.VMEM((ROWS_B, 1), jnp.int32)],
    )(xr, yr, zr, xc, yc, zc)


B_TOT = N * M + N
_NW = 32
_PER_W = B_TOT // _NW
_CH = 544


def _sc_gather(table, idx_all):
    info = plsc.get_sparse_core_info()
    nc = info.num_cores
    mesh = plsc.VectorSubcoreMesh(core_axis_name="c", subcore_axis_name="s")

    @functools.partial(
        pl.kernel, mesh=mesh,
        out_type=jax.ShapeDtypeStruct((B_TOT, D_PAD), jnp.float32),
        scratch_types=[
            pltpu.VMEM((_CH,), jnp.int32),
            pltpu.VMEM((_CH, D_PAD), jnp.float32),
            pltpu.SemaphoreType.DMA,
        ],
    )
    def gk(idx_hbm, tab_hbm, out_hbm, idx_v, rows_v, sem):
        wid = lax.axis_index("s") * nc + lax.axis_index("c")
        for c in range(_PER_W // _CH):
            off = wid * _PER_W + c * _CH
            pltpu.sync_copy(idx_hbm.at[pl.ds(off, _CH)], idx_v)
            pltpu.async_copy(tab_hbm.at[idx_v], rows_v, sem).wait()
            pltpu.sync_copy(rows_v, out_hbm.at[pl.ds(off, _CH)])

    return gk(idx_all, table)


def _attn_body(q_ref, mq_ref, kv_ref, mkv_ref, feat_ref,
               wq_ref, bq_ref, wk_ref, bk_ref, wv_ref, bv_ref,
               wo_ref, bo_ref, wf1_ref, bf1_ref, wf2_ref, bf2_ref,
               lng_ref, lnb_ref, wfu1_ref, bfu1_ref,
               wfu2a_ref, wfu2b_ref, bfu2_ref, out_ref):
    qp = (mq_ref[...] * jnp.dot(q_ref[...], wq_ref[...],
                                preferred_element_type=jnp.float32)
          + bq_ref[...])
    kvf = kv_ref[...].reshape(M * ROWS_D, D_PAD)
    mkv = mkv_ref[...][:, :, None]
    b3 = lambda r: r[...].reshape(1, 1, D_MODEL)
    kp = mkv * jnp.dot(kvf, wk_ref[...],
                       preferred_element_type=jnp.float32).reshape(
                           M, ROWS_D, D_MODEL) + b3(bk_ref)
    vp = mkv * jnp.dot(kvf, wv_ref[...],
                       preferred_element_type=jnp.float32).reshape(
                           M, ROWS_D, D_MODEL) + b3(bv_ref)
    prod = kp * qp[None, :, :]
    lane = lax.broadcasted_iota(jnp.int32, (1, 1, D_MODEL), 2)
    wfull = jnp.zeros((M, ROWS_D, D_MODEL), jnp.float32)
    for h in range(N_HEADS):
        mh = ((lane // HEAD_DIM) == h).astype(jnp.float32)
        s = jnp.sum(prod * mh, axis=-1) * (1.0 / 4.0)
        s = s - jnp.max(s, axis=0, keepdims=True)
        e = jnp.exp(s)
        w = e / jnp.sum(e, axis=0, keepdims=True)
        wfull = wfull + w[:, :, None] * mh
    attn = jnp.sum(wfull * vp, axis=0)
    out = jnp.dot(attn, wo_ref[...], preferred_element_type=jnp.float32) + bo_ref[...]
    ff = jnp.maximum(
        jnp.dot(out, wf1_ref[...], preferred_element_type=jnp.float32)
        + bf1_ref[...], 0.0)
    ff = jnp.dot(ff, wf2_ref[...], preferred_element_type=jnp.float32) + bf2_ref[...]
    t = out + ff
    mu = jnp.mean(t, axis=-1, keepdims=True)
    var = jnp.mean((t - mu) * (t - mu), axis=-1, keepdims=True)
    tgt = (t - mu) * lax.rsqrt(var + 1e-5) * lng_ref[...] + lnb_ref[...]
    tf = jnp.dot(tgt, wfu1_ref[...], preferred_element_type=jnp.float32) + bfu1_ref[...]
    fused = (jnp.dot(feat_ref[...], wfu2a_ref[...],
                     preferred_element_type=jnp.float32)
             + jnp.dot(tf, wfu2b_ref[...], preferred_element_type=jnp.float32)
             + bfu2_ref[...])
    out_ref[...] = fused


def _attention(Q, maskQ, KV3, maskKV, feat, Wq, bq, Wk, bk, Wv, bv, Wo, bo,
               W_f1, b_f1, W_f2, b_f2, ln_g, ln_b, W_fu1, b_fu1,
               W_fu2, b_fu2):
    Wq_p = jnp.pad(Wq, ((0, D_PAD - D_MODEL), (0, 0)))
    Wk_p = jnp.pad(Wk, ((0, D_PAD - D_MODEL), (0, 0)))
    Wv_p = jnp.pad(Wv, ((0, D_PAD - D_MODEL), (0, 0)))
    row2 = lambda w: pl.BlockSpec((ROWS_D, w), lambda i: (i, 0))
    fullw = lambda s: pl.BlockSpec(s, lambda i: (0,) * len(s))
    kvspec = pl.BlockSpec((M, ROWS_D, D_PAD), lambda i: (0, i, 0))
    mkvspec = pl.BlockSpec((M, ROWS_D), lambda i: (0, i))
    return pl.pallas_call(
        _attn_body,
        grid=(N // ROWS_D,),
        out_shape=jax.ShapeDtypeStruct((N, D_CHL), jnp.float32),
        in_specs=[row2(D_PAD), row2(1), kvspec, mkvspec, row2(D_CHL),
                  fullw((D_PAD, D_MODEL)), fullw((1, D_MODEL)),
                  fullw((D_PAD, D_MODEL)), fullw((1, D_MODEL)),
                  fullw((D_PAD, D_MODEL)), fullw((1, D_MODEL)),
                  fullw((D_MODEL, D_MODEL)), fullw((1, D_MODEL)),
                  fullw((D_MODEL, D_FFN)), fullw((1, D_FFN)),
                  fullw((D_FFN, D_MODEL)), fullw((1, D_MODEL)),
                  fullw((1, D_MODEL)), fullw((1, D_MODEL)),
                  fullw((D_MODEL, D_CHL)), fullw((1, D_CHL)),
                  fullw((D_CHL, D_CHL)), fullw((D_CHL, D_CHL)),
                  fullw((1, D_CHL))],
        out_specs=row2(D_CHL),
    )(Q, maskQ, KV3, maskKV, feat,
      Wq_p, bq.reshape(1, -1), Wk_p, bk.reshape(1, -1), Wv_p, bv.reshape(1, -1),
      Wo, bo.reshape(1, -1), W_f1, b_f1.reshape(1, -1), W_f2,
      b_f2.reshape(1, -1), ln_g.reshape(1, -1), ln_b.reshape(1, -1),
      W_fu1, b_fu1.reshape(1, -1), W_fu2[:D_CHL], W_fu2[D_CHL:],
      b_fu2.reshape(1, -1))


def _bn_body(x_ref, g_ref, b_ref, out_ref):
    x = x_ref[...]
    mu = jnp.mean(x, axis=0, keepdims=True)
    var = jnp.mean((x - mu) * (x - mu), axis=0, keepdims=True)
    y = (x - mu) * lax.rsqrt(var + 1e-5) * g_ref[...] + b_ref[...]
    out_ref[...] = jnp.maximum(y, 0.0)


def _batchnorm(x, g, b):
    full = lambda s: pl.BlockSpec(s, lambda: (0,) * len(s))
    return pl.pallas_call(
        _bn_body,
        out_shape=jax.ShapeDtypeStruct((N, D_CHL), jnp.float32),
        in_specs=[full((N, D_CHL)), full((1, D_CHL)), full((1, D_CHL))],
        out_specs=full((N, D_CHL)),
    )(x, g.reshape(1, -1), b.reshape(1, -1))


def kernel(voxel_coords, voxel_features, W_proj, b_proj, W_pe1, b_pe1,
           W_pe2, b_pe2, Wq, bq, Wk, bk, Wv, bv, Wo, bo, W_f1, b_f1,
           W_f2, b_f2, ln_g, ln_b, W_fu1, b_fu1, W_fu2, b_fu2, bn_g, bn_b):
    crd_i32 = voxel_coords.astype(jnp.int32)
    crd_f32 = voxel_coords.astype(jnp.float32)

    src = _compute_src(voxel_features, crd_f32, W_proj, b_proj,
                       W_pe1, b_pe1, W_pe2, b_pe2)
    idx, msk = _topk(crd_i32)

    kv_idx = idx.reshape(M, N)
    kv_msk = msk.reshape(M, N)
    q_msk = msk[:, 0:1]

    idx_all = jnp.concatenate([kv_idx.reshape(-1), idx[:, 0]], axis=0)
    rows = _sc_gather(src, idx_all)
    KV3 = rows[:N * M].reshape(M, N, D_PAD)
    Q = rows[N * M:]

    fused = _attention(Q, q_msk, KV3, kv_msk, voxel_features,
                       Wq, bq, Wk, bk, Wv, bv, Wo, bo, W_f1, b_f1,
                       W_f2, b_f2, ln_g, ln_b, W_fu1, b_fu1, W_fu2, b_fu2)
    return _batchnorm(fused, bn_g, bn_b)
